# Initial kernel scaffold; baseline (speedup 1.0000x reference)
import jax
import jax.numpy as jnp
from jax.experimental import pallas as pl


def kernel(article_bert_embeddings, x, edge_index, article_entity_map_tensor, conv1_W, conv1_b, conv2_W, conv2_b, fc1_W, fc1_b, fc2_W, fc2_b):
    raise NotImplementedError("write your pallas kernel here")


# speedup vs baseline: 8.0548x; 8.0548x over previous
import jax
import jax.numpy as jnp
from jax import lax
from jax.experimental import pallas as pl
from jax.experimental.pallas import tpu as pltpu
from jax.experimental.pallas import tpu_sc as plsc

N = 10000
N16 = 10016
D = 128
NC = 2
NS = 16
NW = NC * NS
CH = 128
NCHUNK = 80
EPT = NCHUNK * CH
E_PAD = NW * EPT
WB_PT = 624
WB_TAIL = N16 - NS * WB_PT
BATCH = 1024
MAX_ENT = 20
BERT = 768
APT = BATCH // NW
IPT = APT * MAX_ENT
ZROW = N

_mesh = plsc.VectorSubcoreMesh(core_axis_name="c", subcore_axis_name="s")

_f32 = jnp.float32


def _zero_span(buf, sp, s):
    base = s * WB_PT
    for q in range(4):
        pltpu.sync_copy(buf, sp.at[pl.ds(base + q * CH, CH)])
    pltpu.sync_copy(buf.at[pl.ds(0, WB_PT - 4 * CH)],
                    sp.at[pl.ds(base + 4 * CH, WB_PT - 4 * CH)])

    @pl.when(s == NS - 1)
    def _tail():
        pltpu.sync_copy(buf.at[pl.ds(0, WB_TAIL)],
                        sp.at[pl.ds(NS * WB_PT, WB_TAIL)])


def _writeback(sp, hbm, c, s):
    pltpu.sync_copy(sp.at[pl.ds(s * WB_PT, WB_PT)],
                    hbm.at[c, pl.ds(s * WB_PT, WB_PT)])

    @pl.when(s == NS - 1)
    def _tail():
        pltpu.sync_copy(sp.at[pl.ds(NS * WB_PT, WB_TAIL)],
                        hbm.at[c, pl.ds(NS * WB_PT, WB_TAIL)])


def _deg_body(col_hbm, deg_hbm, colv, degv):
    c = lax.axis_index("c")
    s = lax.axis_index("s")
    w = c * NS + s

    zeros16 = jnp.zeros((16,), _f32)
    ones16 = jnp.ones((16,), _f32)

    def fill_zero(i, carry):
        degv[pl.ds(i * 16, 16)] = zeros16
        return carry

    lax.fori_loop(0, N16 // 16, fill_zero, 0)
    pltpu.sync_copy(col_hbm.at[w], colv)

    def chunk(j, carry):
        for g in range(CH // 16):
            idx = colv[j, pl.ds(g * 16, 16)]
            plsc.addupdate_scatter(degv, [idx], ones16)
        return carry

    lax.fori_loop(0, NCHUNK, chunk, 0)
    pltpu.sync_copy(degv, deg_hbm.at[w])


_deg_call = pl.kernel(
    _deg_body,
    out_type=jax.ShapeDtypeStruct((NW, N16), _f32),
    mesh=_mesh,
    scratch_types=[
        pltpu.VMEM((NCHUNK, CH), jnp.int32),
        pltpu.VMEM((N16,), _f32),
    ],
    compiler_params=pltpu.CompilerParams(needs_layout_passes=False),
)


def _agg_body(y_hbm, row_hbm, col_hbm, out_hbm, rowv, colv, buf, spagg, sem):
    c = lax.axis_index("c")
    s = lax.axis_index("s")
    w = c * NS + s

    def fill(i, carry):
        for k in range(D // 16):
            buf[i, pl.ds(k * 16, 16)] = jnp.zeros((16,), _f32)
        return carry

    lax.fori_loop(0, CH, fill, 0)
    _zero_span(buf, spagg, s)
    plsc.subcore_barrier()

    pltpu.sync_copy(row_hbm.at[w], rowv)
    pltpu.sync_copy(col_hbm.at[w], colv)

    def chunk(j, carry):
        pltpu.make_async_copy(y_hbm.at[rowv.at[j]], buf, sem).start()
        pltpu.make_async_copy(y_hbm.at[rowv.at[j]], buf, sem).wait()
        pltpu.sync_copy(buf, spagg.at[colv.at[j]], add=True)
        return carry

    lax.fori_loop(0, NCHUNK, chunk, 0)
    plsc.subcore_barrier()
    _writeback(spagg, out_hbm, c, s)


_agg_call = pl.kernel(
    _agg_body,
    out_type=jax.ShapeDtypeStruct((NC, N16, D), _f32),
    mesh=_mesh,
    scratch_types=[
        pltpu.VMEM((NCHUNK, CH), jnp.int32),
        pltpu.VMEM((NCHUNK, CH), jnp.int32),
        pltpu.VMEM((CH, D), _f32),
        pltpu.VMEM_SHARED((N16, D), _f32),
        pltpu.SemaphoreType.DMA,
    ],
)


def _pool_body(ent_hbm, ids_hbm, out_hbm, idxv, safev, rows, outv, sem):
    c = lax.axis_index("c")
    s = lax.axis_index("s")
    w = c * NS + s
    base = w * IPT

    pltpu.sync_copy(ids_hbm.at[pl.ds(base, IPT)], idxv)
    zrow16 = jnp.full((16,), ZROW, jnp.int32)
    for i in range(IPT // 16):
        v = idxv[pl.ds(i * 16, 16)]
        safev[i // 8, pl.ds((i % 8) * 16, 16)] = jnp.where(v >= 0, v, zrow16)
    for q in range(IPT // 128):
        pltpu.make_async_copy(ent_hbm.at[safev.at[q]],
                              rows.at[pl.ds(q * 128, 128)], sem).start()
    for q in range(IPT // 128):
        pltpu.make_async_copy(ent_hbm.at[safev.at[q]],
                              rows.at[pl.ds(q * 128, 128)], sem).wait()

    zeros16 = jnp.zeros((16,), _f32)

    def article(a, carry):
        for k in range(D // 16):
            acc = zeros16
            for e in range(MAX_ENT):
                acc = acc + rows[a * MAX_ENT + e, pl.ds(k * 16, 16)]
            outv[a, pl.ds(k * 16, 16)] = acc
        return carry

    lax.fori_loop(0, APT, article, 0)
    pltpu.sync_copy(outv, out_hbm.at[pl.ds(w * APT, APT)])


_pool_call = pl.kernel(
    _pool_body,
    out_type=jax.ShapeDtypeStruct((BATCH, D), _f32),
    mesh=_mesh,
    scratch_types=[
        pltpu.VMEM((IPT,), jnp.int32),
        pltpu.VMEM((IPT // 128, 128), jnp.int32),
        pltpu.VMEM((IPT, D), _f32),
        pltpu.VMEM((APT, D), _f32),
        pltpu.SemaphoreType.DMA,
    ],
)


_RB = 1024
_NB = 10


def _dinv_from(deg_ref):
    ones = jnp.ones((NW, 1), _f32)
    deg = lax.dot_general(deg_ref[...], ones, (((0,), (0,)), ((), ())),
                          preferred_element_type=_f32)
    return lax.rsqrt(deg + 1.0)


def _b1_body(x_ref, w_ref, deg_ref, y_ref):
    dinv = _dinv_from(deg_ref)
    xw = jnp.dot(x_ref[...], w_ref[...], preferred_element_type=_f32)
    y_ref[...] = xw * dinv


def _b2_body(a_ref, y_ref, deg_ref, w_ref, b_ref, o_ref):
    dinv = _dinv_from(deg_ref)
    h = (a_ref[0] + a_ref[1] + y_ref[...]) * dinv + b_ref[...]
    h = jnp.maximum(h, 0.0)
    o_ref[...] = jnp.dot(h, w_ref[...], preferred_element_type=_f32) * dinv


def _b3_body(a_ref, y_ref, deg_ref, b_ref, o_ref):
    i = pl.program_id(0)
    dinv = _dinv_from(deg_ref)
    h = (a_ref[0] + a_ref[1] + y_ref[...]) * dinv + b_ref[...]
    h = jnp.maximum(h, 0.0)
    rid = i * _RB + lax.broadcasted_iota(jnp.int32, (_RB, 1), 0)
    o_ref[...] = jnp.where(rid < N, h, 0.0)


def _head_body(bert_ref, gnn_ref, ids_ref, w1a_ref, w1b_ref, b1_ref, w2_ref,
               b2_ref, o_ref):
    maskf = (ids_ref[...] != -1).astype(_f32)
    cnt = jnp.sum(maskf, axis=1, keepdims=True)
    gnn = jnp.where(cnt > 0, gnn_ref[...] / jnp.maximum(cnt, 1.0), 0.0)
    z = (jnp.dot(bert_ref[...], w1a_ref[...], preferred_element_type=_f32)
         + jnp.dot(gnn, w1b_ref[...], preferred_element_type=_f32)
         + b1_ref[...])
    z = jnp.maximum(z, 0.0)
    logits = jnp.sum(z * w2_ref[...], axis=1, keepdims=True) + b2_ref[...]
    o_ref[...] = jax.nn.sigmoid(logits)


_b1_call = pl.pallas_call(
    _b1_body,
    grid=(_NB,),
    in_specs=[
        pl.BlockSpec((_RB, D), lambda i: (i, 0)),
        pl.BlockSpec((D, D), lambda i: (0, 0)),
        pl.BlockSpec((NW, _RB), lambda i: (0, i)),
    ],
    out_specs=pl.BlockSpec((_RB, D), lambda i: (i, 0)),
    out_shape=jax.ShapeDtypeStruct((N16, D), _f32),
)

_b2_call = pl.pallas_call(
    _b2_body,
    grid=(_NB,),
    in_specs=[
        pl.BlockSpec((NC, _RB, D), lambda i: (0, i, 0)),
        pl.BlockSpec((_RB, D), lambda i: (i, 0)),
        pl.BlockSpec((NW, _RB), lambda i: (0, i)),
        pl.BlockSpec((D, D), lambda i: (0, 0)),
        pl.BlockSpec((1, D), lambda i: (0, 0)),
    ],
    out_specs=pl.BlockSpec((_RB, D), lambda i: (i, 0)),
    out_shape=jax.ShapeDtypeStruct((N16, D), _f32),
)

_b3_call = pl.pallas_call(
    _b3_body,
    grid=(_NB,),
    in_specs=[
        pl.BlockSpec((NC, _RB, D), lambda i: (0, i, 0)),
        pl.BlockSpec((_RB, D), lambda i: (i, 0)),
        pl.BlockSpec((NW, _RB), lambda i: (0, i)),
        pl.BlockSpec((1, D), lambda i: (0, 0)),
    ],
    out_specs=pl.BlockSpec((_RB, D), lambda i: (i, 0)),
    out_shape=jax.ShapeDtypeStruct((N16, D), _f32),
)

_BB = 256
FC1_OUT = (BERT + D) // 2

_head_call = pl.pallas_call(
    _head_body,
    grid=(BATCH // _BB,),
    in_specs=[
        pl.BlockSpec((_BB, BERT), lambda i: (i, 0)),
        pl.BlockSpec((_BB, D), lambda i: (i, 0)),
        pl.BlockSpec((_BB, MAX_ENT), lambda i: (i, 0)),
        pl.BlockSpec((BERT, FC1_OUT), lambda i: (0, 0)),
        pl.BlockSpec((D, FC1_OUT), lambda i: (0, 0)),
        pl.BlockSpec((1, FC1_OUT), lambda i: (0, 0)),
        pl.BlockSpec((1, FC1_OUT), lambda i: (0, 0)),
        pl.BlockSpec((1, 1), lambda i: (0, 0)),
    ],
    out_specs=pl.BlockSpec((_BB, 1), lambda i: (i, 0)),
    out_shape=jax.ShapeDtypeStruct((BATCH, 1), _f32),
)


def kernel(article_bert_embeddings, x, edge_index, article_entity_map_tensor,
           conv1_W, conv1_b, conv2_W, conv2_b, fc1_W, fc1_b, fc2_W, fc2_b):
    npad = E_PAD - edge_index.shape[1]
    fake = jnp.full((npad,), ZROW, jnp.int32)
    row3 = jnp.concatenate([edge_index[0], fake]).reshape(NW, NCHUNK, CH)
    col3 = jnp.concatenate([edge_index[1], fake]).reshape(NW, NCHUNK, CH)
    ids_flat = article_entity_map_tensor.reshape(-1)

    deg = _deg_call(col3)
    y1 = _b1_call(x, conv1_W, deg)
    agg1 = _agg_call(y1, row3, col3)
    y2 = _b2_call(agg1, y1, deg, conv2_W, conv1_b.reshape(1, D))
    agg2 = _agg_call(y2, row3, col3)
    ent = _b3_call(agg2, y2, deg, conv2_b.reshape(1, D))
    gnn_sums = _pool_call(ent, ids_flat)

    out = _head_call(article_bert_embeddings, gnn_sums,
                     article_entity_map_tensor,
                     fc1_W[:, :BERT].T, fc1_W[:, BERT:].T,
                     fc1_b.reshape(1, FC1_OUT), fc2_W, fc2_b.reshape(1, 1))
    return out


# speedup vs baseline: 20.9305x; 2.5985x over previous
import jax
import jax.numpy as jnp
from jax import lax
from jax.experimental import pallas as pl
from jax.experimental.pallas import tpu as pltpu
from jax.experimental.pallas import tpu_sc as plsc

N = 10000
N16 = 10016
D = 128
NC = 2
NS = 16
NW = NC * NS
E = 320000
CH = 128
NCHUNK = 79
EPT = NCHUNK * CH
RPT = E // NW
PAD_PT = EPT - RPT
WB_PT = 624
WB_TAIL = N16 - NS * WB_PT
BATCH = 1024
MAX_ENT = 20
BERT = 768
APT = BATCH // NW
IPT = APT * MAX_ENT
ZROW = N

_mesh = plsc.VectorSubcoreMesh(core_axis_name="c", subcore_axis_name="s")

_f32 = jnp.float32


def _zero_span(buf, sp, s):
    base = s * WB_PT
    for q in range(4):
        pltpu.sync_copy(buf, sp.at[pl.ds(base + q * CH, CH)])
    pltpu.sync_copy(buf.at[pl.ds(0, WB_PT - 4 * CH)],
                    sp.at[pl.ds(base + 4 * CH, WB_PT - 4 * CH)])

    @pl.when(s == NS - 1)
    def _tail():
        pltpu.sync_copy(buf.at[pl.ds(0, WB_TAIL)],
                        sp.at[pl.ds(NS * WB_PT, WB_TAIL)])


def _writeback(sp, hbm, c, s):
    pltpu.sync_copy(sp.at[pl.ds(s * WB_PT, WB_PT)],
                    hbm.at[c, pl.ds(s * WB_PT, WB_PT)])

    @pl.when(s == NS - 1)
    def _tail():
        pltpu.sync_copy(sp.at[pl.ds(NS * WB_PT, WB_TAIL)],
                        hbm.at[c, pl.ds(NS * WB_PT, WB_TAIL)])


def _deg_body(col_hbm, deg_hbm, colv, degv):
    c = lax.axis_index("c")
    s = lax.axis_index("s")
    w = c * NS + s

    zeros16 = jnp.zeros((16,), _f32)
    ones16 = jnp.ones((16,), _f32)

    def fill_zero(i, carry):
        degv[pl.ds(i * 16, 16)] = zeros16
        return carry

    lax.fori_loop(0, N16 // 16, fill_zero, 0)
    pltpu.sync_copy(col_hbm.at[w], colv)

    def chunk(j, carry):
        for g in range(CH // 16):
            idx = colv[j, pl.ds(g * 16, 16)]
            plsc.addupdate_scatter(degv, [idx], ones16)
        return carry

    lax.fori_loop(0, NCHUNK, chunk, 0)
    pltpu.sync_copy(degv, deg_hbm.at[w])


_deg_call = pl.kernel(
    _deg_body,
    out_type=jax.ShapeDtypeStruct((NW, N16), _f32),
    mesh=_mesh,
    scratch_types=[
        pltpu.VMEM((NCHUNK, CH), jnp.int32),
        pltpu.VMEM((N16,), _f32),
    ],
    compiler_params=pltpu.CompilerParams(needs_layout_passes=False),
)


def _agg_body(y_hbm, row_hbm, col_hbm, out_hbm, rowv, colv, buf, spagg, sem):
    c = lax.axis_index("c")
    s = lax.axis_index("s")
    w = c * NS + s

    def fill(i, carry):
        for k in range(D // 16):
            buf[i, pl.ds(k * 16, 16)] = jnp.zeros((16,), _f32)
        return carry

    lax.fori_loop(0, CH, fill, 0)
    _zero_span(buf, spagg, s)
    plsc.subcore_barrier()

    pltpu.sync_copy(row_hbm.at[w], rowv)
    pltpu.sync_copy(col_hbm.at[w], colv)

    def chunk(j, carry):
        pltpu.make_async_copy(y_hbm.at[rowv.at[j]], buf, sem).start()
        pltpu.make_async_copy(y_hbm.at[rowv.at[j]], buf, sem).wait()
        pltpu.sync_copy(buf, spagg.at[colv.at[j]], add=True)
        return carry

    lax.fori_loop(0, NCHUNK, chunk, 0)
    plsc.subcore_barrier()
    _writeback(spagg, out_hbm, c, s)


_agg_call = pl.kernel(
    _agg_body,
    out_type=jax.ShapeDtypeStruct((NC, N16, D), _f32),
    mesh=_mesh,
    scratch_types=[
        pltpu.VMEM((NCHUNK, CH), jnp.int32),
        pltpu.VMEM((NCHUNK, CH), jnp.int32),
        pltpu.VMEM((CH, D), _f32),
        pltpu.VMEM_SHARED((N16, D), _f32),
        pltpu.SemaphoreType.DMA,
    ],
)


def _pool_body(ent_hbm, ids_hbm, out_hbm, idxv, safev, rows, outv, sem):
    c = lax.axis_index("c")
    s = lax.axis_index("s")
    w = c * NS + s
    base = w * IPT

    pltpu.sync_copy(ids_hbm.at[pl.ds(base, IPT)], idxv)
    zrow16 = jnp.full((16,), ZROW, jnp.int32)
    for i in range(IPT // 16):
        v = idxv[pl.ds(i * 16, 16)]
        safev[i // 8, pl.ds((i % 8) * 16, 16)] = jnp.where(v >= 0, v, zrow16)
    for q in range(IPT // 128):
        pltpu.make_async_copy(ent_hbm.at[safev.at[q]],
                              rows.at[pl.ds(q * 128, 128)], sem).start()
    for q in range(IPT // 128):
        pltpu.make_async_copy(ent_hbm.at[safev.at[q]],
                              rows.at[pl.ds(q * 128, 128)], sem).wait()

    zeros16 = jnp.zeros((16,), _f32)

    def article(a, carry):
        for k in range(D // 16):
            acc = zeros16
            for e in range(MAX_ENT):
                acc = acc + rows[a * MAX_ENT + e, pl.ds(k * 16, 16)]
            outv[a, pl.ds(k * 16, 16)] = acc
        return carry

    lax.fori_loop(0, APT, article, 0)
    pltpu.sync_copy(outv, out_hbm.at[pl.ds(w * APT, APT)])


_pool_call = pl.kernel(
    _pool_body,
    out_type=jax.ShapeDtypeStruct((BATCH, D), _f32),
    mesh=_mesh,
    scratch_types=[
        pltpu.VMEM((IPT,), jnp.int32),
        pltpu.VMEM((IPT // 128, 128), jnp.int32),
        pltpu.VMEM((IPT, D), _f32),
        pltpu.VMEM((APT, D), _f32),
        pltpu.SemaphoreType.DMA,
    ],
)


_RB = 1024
_NB = 10


def _dinv_from(deg_ref):
    ones = jnp.ones((NW, 1), _f32)
    deg = lax.dot_general(deg_ref[...], ones, (((0,), (0,)), ((), ())),
                          preferred_element_type=_f32)
    return lax.rsqrt(deg + 1.0)


def _b1_body(x_ref, w_ref, deg_ref, y_ref):
    dinv = _dinv_from(deg_ref)
    xw = jnp.dot(x_ref[...], w_ref[...], preferred_element_type=_f32)
    y_ref[...] = xw * dinv


def _b2_body(a_ref, y_ref, deg_ref, w_ref, b_ref, o_ref):
    dinv = _dinv_from(deg_ref)
    h = (a_ref[0] + a_ref[1] + y_ref[...]) * dinv + b_ref[...]
    h = jnp.maximum(h, 0.0)
    o_ref[...] = jnp.dot(h, w_ref[...], preferred_element_type=_f32) * dinv


def _b3_body(a_ref, y_ref, deg_ref, b_ref, o_ref):
    i = pl.program_id(0)
    dinv = _dinv_from(deg_ref)
    h = (a_ref[0] + a_ref[1] + y_ref[...]) * dinv + b_ref[...]
    h = jnp.maximum(h, 0.0)
    rid = i * _RB + lax.broadcasted_iota(jnp.int32, (_RB, 1), 0)
    o_ref[...] = jnp.where(rid < N, h, 0.0)


def _head_body(bert_ref, gnn_ref, ids_ref, w1a_ref, w1b_ref, b1_ref, w2_ref,
               b2_ref, o_ref):
    maskf = (ids_ref[...] != -1).astype(_f32)
    cnt = jnp.sum(maskf, axis=1, keepdims=True)
    gnn = jnp.where(cnt > 0, gnn_ref[...] / jnp.maximum(cnt, 1.0), 0.0)
    z = (jnp.dot(bert_ref[...], w1a_ref[...], preferred_element_type=_f32)
         + jnp.dot(gnn, w1b_ref[...], preferred_element_type=_f32)
         + b1_ref[...])
    z = jnp.maximum(z, 0.0)
    logits = jnp.sum(z * w2_ref[...], axis=1, keepdims=True) + b2_ref[...]
    o_ref[...] = jax.nn.sigmoid(logits)


_b1_call = pl.pallas_call(
    _b1_body,
    grid=(_NB,),
    in_specs=[
        pl.BlockSpec((_RB, D), lambda i: (i, 0)),
        pl.BlockSpec((D, D), lambda i: (0, 0)),
        pl.BlockSpec((NW, _RB), lambda i: (0, i)),
    ],
    out_specs=pl.BlockSpec((_RB, D), lambda i: (i, 0)),
    out_shape=jax.ShapeDtypeStruct((N16, D), _f32),
)

_b2_call = pl.pallas_call(
    _b2_body,
    grid=(_NB,),
    in_specs=[
        pl.BlockSpec((NC, _RB, D), lambda i: (0, i, 0)),
        pl.BlockSpec((_RB, D), lambda i: (i, 0)),
        pl.BlockSpec((NW, _RB), lambda i: (0, i)),
        pl.BlockSpec((D, D), lambda i: (0, 0)),
        pl.BlockSpec((1, D), lambda i: (0, 0)),
    ],
    out_specs=pl.BlockSpec((_RB, D), lambda i: (i, 0)),
    out_shape=jax.ShapeDtypeStruct((N16, D), _f32),
)

_b3_call = pl.pallas_call(
    _b3_body,
    grid=(_NB,),
    in_specs=[
        pl.BlockSpec((NC, _RB, D), lambda i: (0, i, 0)),
        pl.BlockSpec((_RB, D), lambda i: (i, 0)),
        pl.BlockSpec((NW, _RB), lambda i: (0, i)),
        pl.BlockSpec((1, D), lambda i: (0, 0)),
    ],
    out_specs=pl.BlockSpec((_RB, D), lambda i: (i, 0)),
    out_shape=jax.ShapeDtypeStruct((N16, D), _f32),
)

_BB = 256
FC1_OUT = (BERT + D) // 2

_head_call = pl.pallas_call(
    _head_body,
    grid=(BATCH // _BB,),
    in_specs=[
        pl.BlockSpec((_BB, BERT), lambda i: (i, 0)),
        pl.BlockSpec((_BB, D), lambda i: (i, 0)),
        pl.BlockSpec((_BB, MAX_ENT), lambda i: (i, 0)),
        pl.BlockSpec((BERT, FC1_OUT), lambda i: (0, 0)),
        pl.BlockSpec((D, FC1_OUT), lambda i: (0, 0)),
        pl.BlockSpec((1, FC1_OUT), lambda i: (0, 0)),
        pl.BlockSpec((1, FC1_OUT), lambda i: (0, 0)),
        pl.BlockSpec((1, 1), lambda i: (0, 0)),
    ],
    out_specs=pl.BlockSpec((_BB, 1), lambda i: (i, 0)),
    out_shape=jax.ShapeDtypeStruct((BATCH, 1), _f32),
)


def kernel(article_bert_embeddings, x, edge_index, article_entity_map_tensor,
           conv1_W, conv1_b, conv2_W, conv2_b, fc1_W, fc1_b, fc2_W, fc2_b):
    fake = jnp.broadcast_to(
        (jnp.arange(PAD_PT, dtype=jnp.int32) % (N16 - N)) + N, (NW, PAD_PT))
    row3 = jnp.concatenate(
        [edge_index[0].reshape(NW, RPT), fake], axis=1).reshape(NW, NCHUNK, CH)
    col3 = jnp.concatenate(
        [edge_index[1].reshape(NW, RPT), fake], axis=1).reshape(NW, NCHUNK, CH)
    ids_flat = article_entity_map_tensor.reshape(-1)

    deg = _deg_call(col3)
    y1 = _b1_call(x, conv1_W, deg)
    agg1 = _agg_call(y1, row3, col3)
    y2 = _b2_call(agg1, y1, deg, conv2_W, conv1_b.reshape(1, D))
    agg2 = _agg_call(y2, row3, col3)
    ent = _b3_call(agg2, y2, deg, conv2_b.reshape(1, D))
    gnn_sums = _pool_call(ent, ids_flat)

    out = _head_call(article_bert_embeddings, gnn_sums,
                     article_entity_map_tensor,
                     fc1_W[:, :BERT].T, fc1_W[:, BERT:].T,
                     fc1_b.reshape(1, FC1_OUT), fc2_W, fc2_b.reshape(1, 1))
    return out


# speedup vs baseline: 29.5117x; 1.4100x over previous
import jax
import jax.numpy as jnp
from jax import lax
from jax.experimental import pallas as pl
from jax.experimental.pallas import tpu as pltpu
from jax.experimental.pallas import tpu_sc as plsc

N = 10000
N16 = 10016
D = 128
NC = 2
NS = 16
NW = NC * NS
E = 320000
CH = 128
NCHUNK = 79
EPT = NCHUNK * CH
RPT = E // NW
PAD_PT = EPT - RPT
WB_PT = 624
WB_TAIL = N16 - NS * WB_PT
BATCH = 1024
MAX_ENT = 20
BERT = 768
APT = BATCH // NW
IPT = APT * MAX_ENT
ZROW = N

_mesh = plsc.VectorSubcoreMesh(core_axis_name="c", subcore_axis_name="s")

_f32 = jnp.float32


def _zero_span(buf, sp, s):
    base = s * WB_PT
    for q in range(4):
        pltpu.sync_copy(buf, sp.at[pl.ds(base + q * CH, CH)])
    pltpu.sync_copy(buf.at[pl.ds(0, WB_PT - 4 * CH)],
                    sp.at[pl.ds(base + 4 * CH, WB_PT - 4 * CH)])

    @pl.when(s == NS - 1)
    def _tail():
        pltpu.sync_copy(buf.at[pl.ds(0, WB_TAIL)],
                        sp.at[pl.ds(NS * WB_PT, WB_TAIL)])


def _writeback(sp, hbm, c, s):
    pltpu.sync_copy(sp.at[pl.ds(s * WB_PT, WB_PT)],
                    hbm.at[c, pl.ds(s * WB_PT, WB_PT)])

    @pl.when(s == NS - 1)
    def _tail():
        pltpu.sync_copy(sp.at[pl.ds(NS * WB_PT, WB_TAIL)],
                        hbm.at[c, pl.ds(NS * WB_PT, WB_TAIL)])


def _deg_body(col_hbm, deg_hbm, colv, degv):
    c = lax.axis_index("c")
    s = lax.axis_index("s")
    w = c * NS + s

    zeros16 = jnp.zeros((16,), _f32)
    ones16 = jnp.ones((16,), _f32)

    def fill_zero(i, carry):
        degv[pl.ds(i * 16, 16)] = zeros16
        return carry

    lax.fori_loop(0, N16 // 16, fill_zero, 0)
    pltpu.sync_copy(col_hbm.at[w], colv)

    def chunk(j, carry):
        for g in range(CH // 16):
            idx = colv[j, pl.ds(g * 16, 16)]
            plsc.addupdate_scatter(degv, [idx], ones16)
        return carry

    lax.fori_loop(0, NCHUNK, chunk, 0)
    pltpu.sync_copy(degv, deg_hbm.at[w])


_deg_call = pl.kernel(
    _deg_body,
    out_type=jax.ShapeDtypeStruct((NW, N16), _f32),
    mesh=_mesh,
    scratch_types=[
        pltpu.VMEM((NCHUNK, CH), jnp.int32),
        pltpu.VMEM((N16,), _f32),
    ],
    compiler_params=pltpu.CompilerParams(needs_layout_passes=False),
)


def _agg_body(y_hbm, pk_hbm, out_hbm, pkv, buf0, buf1, idxs, spagg, sem0, sem1):
    c = lax.axis_index("c")
    s = lax.axis_index("s")
    w = c * NS + s
    bufs = (buf0, buf1)
    sems = (sem0, sem1)

    def fill(i, carry):
        for k in range(D // 16):
            buf0[i, pl.ds(k * 16, 16)] = jnp.zeros((16,), _f32)
        return carry

    lax.fori_loop(0, CH, fill, 0)
    _zero_span(buf0, spagg, s)
    plsc.subcore_barrier()

    pltpu.sync_copy(pk_hbm.at[w], pkv)

    def unpack_row(j, b):
        for g in range(CH // 16):
            v = pkv[j, pl.ds(g * 16, 16)]
            idxs[b, pl.ds(g * 16, 16)] = v & 0xFFFF

    def unpack_col(j, b):
        for g in range(CH // 16):
            v = pkv[j, pl.ds(g * 16, 16)]
            idxs[2 + b, pl.ds(g * 16, 16)] = lax.shift_right_logical(v, 16)

    def start_gather(j, b):
        pltpu.make_async_copy(y_hbm.at[idxs.at[b]], bufs[b], sems[b]).start()

    def finish_chunk(j, b):
        pltpu.make_async_copy(y_hbm.at[idxs.at[b]], bufs[b], sems[b]).wait()
        unpack_col(j, b)
        pltpu.sync_copy(bufs[b], spagg.at[idxs.at[2 + b]], add=True)

    for b in range(2):
        unpack_row(b, b)
        start_gather(b, b)

    def pair(t, carry):
        j0 = 2 * t
        for b in range(2):
            j = j0 + b
            finish_chunk(j, b)
            unpack_row(j + 2, b)
            start_gather(j + 2, b)
        return carry

    lax.fori_loop(0, (NCHUNK - 3) // 2, pair, 0)
    finish_chunk(NCHUNK - 3, 0)
    unpack_row(NCHUNK - 1, 0)
    start_gather(NCHUNK - 1, 0)
    finish_chunk(NCHUNK - 2, 1)
    finish_chunk(NCHUNK - 1, 0)

    plsc.subcore_barrier()
    _writeback(spagg, out_hbm, c, s)


_agg_call = pl.kernel(
    _agg_body,
    out_type=jax.ShapeDtypeStruct((NC, N16, D), _f32),
    mesh=_mesh,
    scratch_types=[
        pltpu.VMEM((NCHUNK, CH), jnp.int32),
        pltpu.VMEM((CH, D), _f32),
        pltpu.VMEM((CH, D), _f32),
        pltpu.VMEM((8, CH), jnp.int32),
        pltpu.VMEM_SHARED((N16, D), _f32),
        pltpu.SemaphoreType.DMA,
        pltpu.SemaphoreType.DMA,
    ],
    compiler_params=pltpu.CompilerParams(needs_layout_passes=False),
)


def _pool_body(ent_hbm, ids_hbm, out_hbm, idxv, safev, rows, outv, sem):
    c = lax.axis_index("c")
    s = lax.axis_index("s")
    w = c * NS + s
    base = w * IPT

    pltpu.sync_copy(ids_hbm.at[pl.ds(base, IPT)], idxv)
    zrow16 = jnp.full((16,), ZROW, jnp.int32)
    for i in range(IPT // 16):
        v = idxv[pl.ds(i * 16, 16)]
        safev[i // 8, pl.ds((i % 8) * 16, 16)] = jnp.where(v >= 0, v, zrow16)
    for q in range(IPT // 128):
        pltpu.make_async_copy(ent_hbm.at[safev.at[q]],
                              rows.at[pl.ds(q * 128, 128)], sem).start()
    for q in range(IPT // 128):
        pltpu.make_async_copy(ent_hbm.at[safev.at[q]],
                              rows.at[pl.ds(q * 128, 128)], sem).wait()

    zeros16 = jnp.zeros((16,), _f32)

    def article(a, carry):
        for k in range(D // 16):
            acc = zeros16
            for e in range(MAX_ENT):
                acc = acc + rows[a * MAX_ENT + e, pl.ds(k * 16, 16)]
            outv[a, pl.ds(k * 16, 16)] = acc
        return carry

    lax.fori_loop(0, APT, article, 0)
    pltpu.sync_copy(outv, out_hbm.at[pl.ds(w * APT, APT)])


_pool_call = pl.kernel(
    _pool_body,
    out_type=jax.ShapeDtypeStruct((BATCH, D), _f32),
    mesh=_mesh,
    scratch_types=[
        pltpu.VMEM((IPT,), jnp.int32),
        pltpu.VMEM((IPT // 128, 128), jnp.int32),
        pltpu.VMEM((IPT, D), _f32),
        pltpu.VMEM((APT, D), _f32),
        pltpu.SemaphoreType.DMA,
    ],
)


_RB = 1024
_NB = 10


def _dinv_from(deg_ref):
    ones = jnp.ones((NW, 1), _f32)
    deg = lax.dot_general(deg_ref[...], ones, (((0,), (0,)), ((), ())),
                          preferred_element_type=_f32)
    return lax.rsqrt(deg + 1.0)


def _b1_body(x_ref, w_ref, deg_ref, y_ref):
    dinv = _dinv_from(deg_ref)
    xw = jnp.dot(x_ref[...], w_ref[...], preferred_element_type=_f32)
    y_ref[...] = xw * dinv


def _b2_body(a_ref, y_ref, deg_ref, w_ref, b_ref, o_ref):
    dinv = _dinv_from(deg_ref)
    h = (a_ref[0] + a_ref[1] + y_ref[...]) * dinv + b_ref[...]
    h = jnp.maximum(h, 0.0)
    o_ref[...] = jnp.dot(h, w_ref[...], preferred_element_type=_f32) * dinv


def _b3_body(a_ref, y_ref, deg_ref, b_ref, o_ref):
    i = pl.program_id(0)
    dinv = _dinv_from(deg_ref)
    h = (a_ref[0] + a_ref[1] + y_ref[...]) * dinv + b_ref[...]
    h = jnp.maximum(h, 0.0)
    rid = i * _RB + lax.broadcasted_iota(jnp.int32, (_RB, 1), 0)
    o_ref[...] = jnp.where(rid < N, h, 0.0)


def _head_body(bert_ref, gnn_ref, ids_ref, w1a_ref, w1b_ref, b1_ref, w2_ref,
               b2_ref, o_ref):
    maskf = (ids_ref[...] != -1).astype(_f32)
    cnt = jnp.sum(maskf, axis=1, keepdims=True)
    gnn = jnp.where(cnt > 0, gnn_ref[...] / jnp.maximum(cnt, 1.0), 0.0)
    z = (jnp.dot(bert_ref[...], w1a_ref[...], preferred_element_type=_f32)
         + jnp.dot(gnn, w1b_ref[...], preferred_element_type=_f32)
         + b1_ref[...])
    z = jnp.maximum(z, 0.0)
    logits = jnp.sum(z * w2_ref[...], axis=1, keepdims=True) + b2_ref[...]
    o_ref[...] = jax.nn.sigmoid(logits)


_b1_call = pl.pallas_call(
    _b1_body,
    grid=(_NB,),
    in_specs=[
        pl.BlockSpec((_RB, D), lambda i: (i, 0)),
        pl.BlockSpec((D, D), lambda i: (0, 0)),
        pl.BlockSpec((NW, _RB), lambda i: (0, i)),
    ],
    out_specs=pl.BlockSpec((_RB, D), lambda i: (i, 0)),
    out_shape=jax.ShapeDtypeStruct((N16, D), _f32),
)

_b2_call = pl.pallas_call(
    _b2_body,
    grid=(_NB,),
    in_specs=[
        pl.BlockSpec((NC, _RB, D), lambda i: (0, i, 0)),
        pl.BlockSpec((_RB, D), lambda i: (i, 0)),
        pl.BlockSpec((NW, _RB), lambda i: (0, i)),
        pl.BlockSpec((D, D), lambda i: (0, 0)),
        pl.BlockSpec((1, D), lambda i: (0, 0)),
    ],
    out_specs=pl.BlockSpec((_RB, D), lambda i: (i, 0)),
    out_shape=jax.ShapeDtypeStruct((N16, D), _f32),
)

_b3_call = pl.pallas_call(
    _b3_body,
    grid=(_NB,),
    in_specs=[
        pl.BlockSpec((NC, _RB, D), lambda i: (0, i, 0)),
        pl.BlockSpec((_RB, D), lambda i: (i, 0)),
        pl.BlockSpec((NW, _RB), lambda i: (0, i)),
        pl.BlockSpec((1, D), lambda i: (0, 0)),
    ],
    out_specs=pl.BlockSpec((_RB, D), lambda i: (i, 0)),
    out_shape=jax.ShapeDtypeStruct((N16, D), _f32),
)

_BB = 256
FC1_OUT = (BERT + D) // 2

_head_call = pl.pallas_call(
    _head_body,
    grid=(BATCH // _BB,),
    in_specs=[
        pl.BlockSpec((_BB, BERT), lambda i: (i, 0)),
        pl.BlockSpec((_BB, D), lambda i: (i, 0)),
        pl.BlockSpec((_BB, MAX_ENT), lambda i: (i, 0)),
        pl.BlockSpec((BERT, FC1_OUT), lambda i: (0, 0)),
        pl.BlockSpec((D, FC1_OUT), lambda i: (0, 0)),
        pl.BlockSpec((1, FC1_OUT), lambda i: (0, 0)),
        pl.BlockSpec((1, FC1_OUT), lambda i: (0, 0)),
        pl.BlockSpec((1, 1), lambda i: (0, 0)),
    ],
    out_specs=pl.BlockSpec((_BB, 1), lambda i: (i, 0)),
    out_shape=jax.ShapeDtypeStruct((BATCH, 1), _f32),
)


def kernel(article_bert_embeddings, x, edge_index, article_entity_map_tensor,
           conv1_W, conv1_b, conv2_W, conv2_b, fc1_W, fc1_b, fc2_W, fc2_b):
    fake = jnp.broadcast_to(
        (jnp.arange(PAD_PT, dtype=jnp.int32) % (N16 - N)) + N, (NW, PAD_PT))
    row3 = jnp.concatenate(
        [edge_index[0].reshape(NW, RPT), fake], axis=1).reshape(NW, NCHUNK, CH)
    col3 = jnp.concatenate(
        [edge_index[1].reshape(NW, RPT), fake], axis=1).reshape(NW, NCHUNK, CH)
    pk3 = jnp.bitwise_or(jnp.left_shift(col3, 16), row3)
    ids_flat = article_entity_map_tensor.reshape(-1)

    deg = _deg_call(col3)
    y1 = _b1_call(x, conv1_W, deg)
    agg1 = _agg_call(y1, pk3)
    y2 = _b2_call(agg1, y1, deg, conv2_W, conv1_b.reshape(1, D))
    agg2 = _agg_call(y2, pk3)
    ent = _b3_call(agg2, y2, deg, conv2_b.reshape(1, D))
    gnn_sums = _pool_call(ent, ids_flat)

    out = _head_call(article_bert_embeddings, gnn_sums,
                     article_entity_map_tensor,
                     fc1_W[:, :BERT].T, fc1_W[:, BERT:].T,
                     fc1_b.reshape(1, FC1_OUT), fc2_W, fc2_b.reshape(1, 1))
    return out


# speedup vs baseline: 30.7000x; 1.0403x over previous
import jax
import jax.numpy as jnp
from jax import lax
from jax.experimental import pallas as pl
from jax.experimental.pallas import tpu as pltpu
from jax.experimental.pallas import tpu_sc as plsc

N = 10000
N16 = 10016
D = 128
NC = 2
NS = 16
NW = NC * NS
E = 320000
CH = 128
NCHUNK = 79
EPT = NCHUNK * CH
RPT = E // NW
PAD_PT = EPT - RPT
WB_PT = 624
WB_TAIL = N16 - NS * WB_PT
BATCH = 1024
MAX_ENT = 20
BERT = 768
APT = BATCH // NW
IPT = APT * MAX_ENT
ZROW = N

_mesh = plsc.VectorSubcoreMesh(core_axis_name="c", subcore_axis_name="s")

_f32 = jnp.float32


def _zero_span(buf, sp, s):
    base = s * WB_PT
    for q in range(4):
        pltpu.sync_copy(buf, sp.at[pl.ds(base + q * CH, CH)])
    pltpu.sync_copy(buf.at[pl.ds(0, WB_PT - 4 * CH)],
                    sp.at[pl.ds(base + 4 * CH, WB_PT - 4 * CH)])

    @pl.when(s == NS - 1)
    def _tail():
        pltpu.sync_copy(buf.at[pl.ds(0, WB_TAIL)],
                        sp.at[pl.ds(NS * WB_PT, WB_TAIL)])


def _writeback(sp, hbm, c, s):
    pltpu.sync_copy(sp.at[pl.ds(s * WB_PT, WB_PT)],
                    hbm.at[c, pl.ds(s * WB_PT, WB_PT)])

    @pl.when(s == NS - 1)
    def _tail():
        pltpu.sync_copy(sp.at[pl.ds(NS * WB_PT, WB_TAIL)],
                        hbm.at[c, pl.ds(NS * WB_PT, WB_TAIL)])


def _deg_body(col_hbm, deg_hbm, colv, degv):
    c = lax.axis_index("c")
    s = lax.axis_index("s")
    w = c * NS + s

    zeros16 = jnp.zeros((16,), _f32)
    ones16 = jnp.ones((16,), _f32)

    def fill_zero(i, carry):
        degv[pl.ds(i * 16, 16)] = zeros16
        return carry

    lax.fori_loop(0, N16 // 16, fill_zero, 0)
    pltpu.sync_copy(col_hbm.at[w], colv)

    def chunk(j, carry):
        for g in range(CH // 16):
            idx = colv[j, pl.ds(g * 16, 16)]
            plsc.addupdate_scatter(degv, [idx], ones16)
        return carry

    lax.fori_loop(0, NCHUNK, chunk, 0)
    pltpu.sync_copy(degv, deg_hbm.at[w])


_deg_call = pl.kernel(
    _deg_body,
    out_type=jax.ShapeDtypeStruct((NW, N16), _f32),
    mesh=_mesh,
    scratch_types=[
        pltpu.VMEM((NCHUNK, CH), jnp.int32),
        pltpu.VMEM((N16,), _f32),
    ],
    compiler_params=pltpu.CompilerParams(needs_layout_passes=False),
)


def _agg_body(y_hbm, pk_hbm, out_hbm, pkv, buf0, buf1, idxs, spagg, sem0, sem1):
    c = lax.axis_index("c")
    s = lax.axis_index("s")
    w = c * NS + s
    bufs = (buf0, buf1)
    sems = (sem0, sem1)

    def fill(i, carry):
        for k in range(D // 16):
            buf0[i, pl.ds(k * 16, 16)] = jnp.zeros((16,), _f32)
        return carry

    lax.fori_loop(0, CH, fill, 0)
    _zero_span(buf0, spagg, s)
    plsc.subcore_barrier()

    pltpu.sync_copy(pk_hbm.at[w], pkv)

    def unpack_row(j, b):
        for g in range(CH // 16):
            v = pkv[j, pl.ds(g * 16, 16)]
            idxs[b, pl.ds(g * 16, 16)] = v & 0xFFFF

    def unpack_col(j, b):
        for g in range(CH // 16):
            v = pkv[j, pl.ds(g * 16, 16)]
            idxs[2 + b, pl.ds(g * 16, 16)] = lax.shift_right_logical(v, 16)

    def start_gather(j, b):
        pltpu.make_async_copy(y_hbm.at[idxs.at[b]], bufs[b], sems[b]).start()

    def finish_chunk(j, b):
        pltpu.make_async_copy(y_hbm.at[idxs.at[b]], bufs[b], sems[b]).wait()
        unpack_col(j, b)
        pltpu.sync_copy(bufs[b], spagg.at[idxs.at[2 + b]], add=True)

    for b in range(2):
        unpack_row(b, b)
        start_gather(b, b)

    def pair(t, carry):
        j0 = 2 * t
        for b in range(2):
            j = j0 + b
            finish_chunk(j, b)
            unpack_row(j + 2, b)
            start_gather(j + 2, b)
        return carry

    lax.fori_loop(0, (NCHUNK - 3) // 2, pair, 0)
    finish_chunk(NCHUNK - 3, 0)
    unpack_row(NCHUNK - 1, 0)
    start_gather(NCHUNK - 1, 0)
    finish_chunk(NCHUNK - 2, 1)
    finish_chunk(NCHUNK - 1, 0)

    plsc.subcore_barrier()
    _writeback(spagg, out_hbm, c, s)


_agg_call = pl.kernel(
    _agg_body,
    out_type=jax.ShapeDtypeStruct((NC, N16, D), _f32),
    mesh=_mesh,
    scratch_types=[
        pltpu.VMEM((NCHUNK, CH), jnp.int32),
        pltpu.VMEM((CH, D), _f32),
        pltpu.VMEM((CH, D), _f32),
        pltpu.VMEM((8, CH), jnp.int32),
        pltpu.VMEM_SHARED((N16, D), _f32),
        pltpu.SemaphoreType.DMA,
        pltpu.SemaphoreType.DMA,
    ],
    compiler_params=pltpu.CompilerParams(needs_layout_passes=False),
)


def _pool_body(ent_hbm, ids_hbm, out_hbm, idxv, safev, rows, outv, sem):
    c = lax.axis_index("c")
    s = lax.axis_index("s")
    w = c * NS + s
    base = w * IPT

    pltpu.sync_copy(ids_hbm.at[pl.ds(base, IPT)], idxv)
    zrow16 = jnp.full((16,), ZROW, jnp.int32)
    for i in range(IPT // 16):
        v = idxv[pl.ds(i * 16, 16)]
        safev[i // 8, pl.ds((i % 8) * 16, 16)] = jnp.where(v >= 0, v, zrow16)
    for q in range(IPT // 128):
        pltpu.make_async_copy(ent_hbm.at[safev.at[q]],
                              rows.at[pl.ds(q * 128, 128)], sem).start()
    for q in range(IPT // 128):
        pltpu.make_async_copy(ent_hbm.at[safev.at[q]],
                              rows.at[pl.ds(q * 128, 128)], sem).wait()

    zeros16 = jnp.zeros((16,), _f32)

    def article(a, carry):
        for k in range(D // 16):
            acc = zeros16
            for e in range(MAX_ENT):
                acc = acc + rows[a * MAX_ENT + e, pl.ds(k * 16, 16)]
            outv[a, pl.ds(k * 16, 16)] = acc
        return carry

    lax.fori_loop(0, APT, article, 0)
    pltpu.sync_copy(outv, out_hbm.at[pl.ds(w * APT, APT)])


_pool_call = pl.kernel(
    _pool_body,
    out_type=jax.ShapeDtypeStruct((BATCH, D), _f32),
    mesh=_mesh,
    scratch_types=[
        pltpu.VMEM((IPT,), jnp.int32),
        pltpu.VMEM((IPT // 128, 128), jnp.int32),
        pltpu.VMEM((IPT, D), _f32),
        pltpu.VMEM((APT, D), _f32),
        pltpu.SemaphoreType.DMA,
    ],
)


_RB = 2048
_NB = 5


def _dinv_from(deg_ref):
    ones = jnp.ones((NW, 1), _f32)
    deg = lax.dot_general(deg_ref[...], ones, (((0,), (0,)), ((), ())),
                          preferred_element_type=_f32)
    return lax.rsqrt(deg + 1.0)


def _b1_body(x_ref, w_ref, deg_ref, y_ref):
    dinv = _dinv_from(deg_ref)
    xw = jnp.dot(x_ref[...], w_ref[...], preferred_element_type=_f32)
    y_ref[...] = xw * dinv


def _b2_body(a_ref, y_ref, deg_ref, w_ref, b_ref, o_ref):
    dinv = _dinv_from(deg_ref)
    h = (a_ref[0] + a_ref[1] + y_ref[...]) * dinv + b_ref[...]
    h = jnp.maximum(h, 0.0)
    o_ref[...] = jnp.dot(h, w_ref[...], preferred_element_type=_f32) * dinv


def _b3_body(a_ref, y_ref, deg_ref, b_ref, o_ref):
    i = pl.program_id(0)
    dinv = _dinv_from(deg_ref)
    h = (a_ref[0] + a_ref[1] + y_ref[...]) * dinv + b_ref[...]
    h = jnp.maximum(h, 0.0)
    rid = i * _RB + lax.broadcasted_iota(jnp.int32, (_RB, 1), 0)
    o_ref[...] = jnp.where(rid < N, h, 0.0)


def _head_body(bert_ref, gnn_ref, ids_ref, w1a_ref, w1b_ref, b1_ref, w2_ref,
               b2_ref, o_ref):
    maskf = (ids_ref[...] != -1).astype(_f32)
    cnt = jnp.sum(maskf, axis=1, keepdims=True)
    gnn = jnp.where(cnt > 0, gnn_ref[...] / jnp.maximum(cnt, 1.0), 0.0)
    z = (jnp.dot(bert_ref[...], w1a_ref[...], preferred_element_type=_f32)
         + jnp.dot(gnn, w1b_ref[...], preferred_element_type=_f32)
         + b1_ref[...])
    z = jnp.maximum(z, 0.0)
    logits = jnp.sum(z * w2_ref[...], axis=1, keepdims=True) + b2_ref[...]
    o_ref[...] = jax.nn.sigmoid(logits)


_b1_call = pl.pallas_call(
    _b1_body,
    grid=(_NB,),
    in_specs=[
        pl.BlockSpec((_RB, D), lambda i: (i, 0)),
        pl.BlockSpec((D, D), lambda i: (0, 0)),
        pl.BlockSpec((NW, _RB), lambda i: (0, i)),
    ],
    out_specs=pl.BlockSpec((_RB, D), lambda i: (i, 0)),
    out_shape=jax.ShapeDtypeStruct((N16, D), _f32),
)

_b2_call = pl.pallas_call(
    _b2_body,
    grid=(_NB,),
    in_specs=[
        pl.BlockSpec((NC, _RB, D), lambda i: (0, i, 0)),
        pl.BlockSpec((_RB, D), lambda i: (i, 0)),
        pl.BlockSpec((NW, _RB), lambda i: (0, i)),
        pl.BlockSpec((D, D), lambda i: (0, 0)),
        pl.BlockSpec((1, D), lambda i: (0, 0)),
    ],
    out_specs=pl.BlockSpec((_RB, D), lambda i: (i, 0)),
    out_shape=jax.ShapeDtypeStruct((N16, D), _f32),
)

_b3_call = pl.pallas_call(
    _b3_body,
    grid=(_NB,),
    in_specs=[
        pl.BlockSpec((NC, _RB, D), lambda i: (0, i, 0)),
        pl.BlockSpec((_RB, D), lambda i: (i, 0)),
        pl.BlockSpec((NW, _RB), lambda i: (0, i)),
        pl.BlockSpec((1, D), lambda i: (0, 0)),
    ],
    out_specs=pl.BlockSpec((_RB, D), lambda i: (i, 0)),
    out_shape=jax.ShapeDtypeStruct((N16, D), _f32),
)

_BB = 512
FC1_OUT = (BERT + D) // 2

_head_call = pl.pallas_call(
    _head_body,
    grid=(BATCH // _BB,),
    in_specs=[
        pl.BlockSpec((_BB, BERT), lambda i: (i, 0)),
        pl.BlockSpec((_BB, D), lambda i: (i, 0)),
        pl.BlockSpec((_BB, MAX_ENT), lambda i: (i, 0)),
        pl.BlockSpec((BERT, FC1_OUT), lambda i: (0, 0)),
        pl.BlockSpec((D, FC1_OUT), lambda i: (0, 0)),
        pl.BlockSpec((1, FC1_OUT), lambda i: (0, 0)),
        pl.BlockSpec((1, FC1_OUT), lambda i: (0, 0)),
        pl.BlockSpec((1, 1), lambda i: (0, 0)),
    ],
    out_specs=pl.BlockSpec((_BB, 1), lambda i: (i, 0)),
    out_shape=jax.ShapeDtypeStruct((BATCH, 1), _f32),
)


def kernel(article_bert_embeddings, x, edge_index, article_entity_map_tensor,
           conv1_W, conv1_b, conv2_W, conv2_b, fc1_W, fc1_b, fc2_W, fc2_b):
    fake = jnp.broadcast_to(
        (jnp.arange(PAD_PT, dtype=jnp.int32) % (N16 - N)) + N, (NW, PAD_PT))
    row3 = jnp.concatenate(
        [edge_index[0].reshape(NW, RPT), fake], axis=1).reshape(NW, NCHUNK, CH)
    col3 = jnp.concatenate(
        [edge_index[1].reshape(NW, RPT), fake], axis=1).reshape(NW, NCHUNK, CH)
    pk3 = jnp.bitwise_or(jnp.left_shift(col3, 16), row3)
    ids_flat = article_entity_map_tensor.reshape(-1)

    deg = _deg_call(col3)
    y1 = _b1_call(x, conv1_W, deg)
    agg1 = _agg_call(y1, pk3)
    y2 = _b2_call(agg1, y1, deg, conv2_W, conv1_b.reshape(1, D))
    agg2 = _agg_call(y2, pk3)
    ent = _b3_call(agg2, y2, deg, conv2_b.reshape(1, D))
    gnn_sums = _pool_call(ent, ids_flat)

    out = _head_call(article_bert_embeddings, gnn_sums,
                     article_entity_map_tensor,
                     fc1_W[:, :BERT].T, fc1_W[:, BERT:].T,
                     fc1_b.reshape(1, FC1_OUT), fc2_W, fc2_b.reshape(1, 1))
    return out
